# baseline (speedup 1.0000x reference)
import functools

import jax
import jax.numpy as jnp
import numpy as np
from jax import lax
from jax.experimental import pallas as pl
from jax.experimental.pallas import tpu as pltpu
from jax.experimental.pallas import tpu_sc as plsc

N_NODES = 10000
N_PAD = 10240
E_EDGES = 160000
E_PAD = 163840
D_IN = 256
D_HALF = 128
D_H = 512
FC_DIM = 1024
N_HEADS = 8
HEAD_DIM = 64

NC = 2
NS = 16
BATCH = 128
NBATCH = E_PAD // (NS * BATCH)
CB = 8
NCHUNK = NBATCH // CB
CB2 = 40
NCHUNK2 = NBATCH // CB2
ROWS_PER_TILE = N_PAD // NS

R_BLK = 1024
NB = N_PAD // R_BLK


def _deg_body(dst_s, dst_t, ones_hbm, zeros16, deg_out, idx_v, ones_v, acc):
    c = lax.axis_index("c")
    s = lax.axis_index("s")
    rows = pl.ds(s * ROWS_PER_TILE, ROWS_PER_TILE)
    pltpu.sync_copy(ones_hbm, ones_v)
    pltpu.sync_copy(zeros16.at[rows], acc.at[rows])
    plsc.subcore_barrier()

    def count(slab):
        def ch_body(ch, carry):
            pltpu.sync_copy(slab.at[s, ch], idx_v)
            for b in range(CB):
                pltpu.sync_copy(ones_v, acc.at[idx_v.at[b]], add=True)
            return carry

        lax.fori_loop(0, NCHUNK, ch_body, 0)

    @pl.when(c == 0)
    def _():
        count(dst_s)

    @pl.when(c == 1)
    def _():
        count(dst_t)

    plsc.subcore_barrier()

    @pl.when(c == 0)
    def _():
        pltpu.sync_copy(acc.at[rows], deg_out.at[0, rows])

    @pl.when(c == 1)
    def _():
        pltpu.sync_copy(acc.at[rows], deg_out.at[1, rows])


def _run_deg(dst3_s, dst3_t, ones128, zeros16):
    return pl.kernel(
        _deg_body,
        out_type=jax.ShapeDtypeStruct((2, N_PAD, 16), jnp.float32),
        mesh=plsc.VectorSubcoreMesh(core_axis_name="c", subcore_axis_name="s"),
        compiler_params=pltpu.CompilerParams(needs_layout_passes=False, use_tc_tiling_on_sc=False),
        scratch_types=[
            pltpu.VMEM((CB, BATCH), jnp.int32),
            pltpu.VMEM((BATCH, 16), jnp.float32),
            pltpu.VMEM_SHARED((N_PAD, 16), jnp.float32),
        ],
    )(dst3_s, dst3_t, ones128, zeros16)


def _norm_g_body(deg_ref, x_ref, norm_ref, g_ref, norm2d_ref):
    deg = deg_ref[...]
    norm = lax.rsqrt(deg + 1.0)
    norm_ref[...] = norm
    g_ref[...] = x_ref[...] * norm[:, 0:1]
    norm2d_ref[...] = norm[:, 0].reshape(R_BLK // 128, 128)


def _norm1d_body(deg2d_ref, norm2d_ref):
    norm2d_ref[...] = lax.rsqrt(deg2d_ref[...] + 1.0)


def _run_norm1d(deg2d):
    return pl.pallas_call(
        _norm1d_body,
        out_shape=jax.ShapeDtypeStruct(deg2d.shape, jnp.float32),
    )(deg2d)


def _run_norm_g(deg16g, xpad):
    return pl.pallas_call(
        _norm_g_body,
        grid=(NB,),
        in_specs=[
            pl.BlockSpec((R_BLK, 16), lambda i: (i, 0)),
            pl.BlockSpec((R_BLK, D_IN), lambda i: (i, 0)),
        ],
        out_specs=[
            pl.BlockSpec((R_BLK, 16), lambda i: (i, 0)),
            pl.BlockSpec((R_BLK, D_IN), lambda i: (i, 0)),
            pl.BlockSpec((R_BLK // 128, 128), lambda i: (i, 0)),
        ],
        out_shape=[
            jax.ShapeDtypeStruct((N_PAD, 16), jnp.float32),
            jax.ShapeDtypeStruct((N_PAD, D_IN), jnp.float32),
            jax.ShapeDtypeStruct((N_PAD // 128, 128), jnp.float32),
        ],
    )(deg16g, xpad)


def _ssum_body(src3_s, dst3_s, src3_t, dst3_t, n2d_s, n2d_t, z16,
               ss_s, ss_t, isrc, idst, nbuf, nvm, acc16):
    s = lax.axis_index("s")
    c = lax.axis_index("c")
    rows = pl.ds(s * ROWS_PER_TILE, ROWS_PER_TILE)

    def ssum_phase(srcslab, dstslab, n2d, out):
        pltpu.sync_copy(n2d, nvm)
        pltpu.sync_copy(z16.at[pl.ds(0, BATCH)], nbuf)
        pltpu.sync_copy(z16.at[rows], acc16.at[rows])
        plsc.subcore_barrier()
        lane = lax.iota(jnp.int32, 16)
        col0 = jnp.zeros((16,), jnp.int32)

        def ch_body(ch, carry):
            pltpu.sync_copy(srcslab.at[s, ch], isrc)
            pltpu.sync_copy(dstslab.at[s, ch], idst)
            for b in range(CB):
                for k in range(BATCH // 16):
                    dvec = idst[b, pl.ds(k * 16, 16)]
                    vals = plsc.load_gather(
                        nvm, [lax.shift_right_logical(dvec, 7),
                              lax.bitwise_and(dvec, 127)])
                    plsc.store_scatter(nbuf, [lane + (k * 16), col0], vals)
                pltpu.sync_copy(nbuf, acc16.at[isrc.at[b]], add=True)
            return carry

        lax.fori_loop(0, NCHUNK, ch_body, 0)
        plsc.subcore_barrier()
        pltpu.sync_copy(acc16.at[rows], out.at[rows])

    @pl.when(c == 0)
    def _():
        ssum_phase(src3_s, dst3_s, n2d_s, ss_s)

    @pl.when(c == 1)
    def _():
        ssum_phase(src3_t, dst3_t, n2d_t, ss_t)


def _run_ssum(src3_s, dst3_s, src3_t, dst3_t, n2d_s, n2d_t, z16):
    s_sds = jax.ShapeDtypeStruct((N_PAD, 16), jnp.float32)
    return pl.kernel(
        _ssum_body,
        out_type=[s_sds, s_sds],
        mesh=plsc.VectorSubcoreMesh(core_axis_name="c", subcore_axis_name="s"),
        compiler_params=pltpu.CompilerParams(needs_layout_passes=False, use_tc_tiling_on_sc=False),
        scratch_types=[
            pltpu.VMEM((CB, BATCH), jnp.int32),
            pltpu.VMEM((CB, BATCH), jnp.int32),
            pltpu.VMEM((BATCH, 16), jnp.float32),
            pltpu.VMEM((N_PAD // 128, 128), jnp.float32),
            pltpu.VMEM_SHARED((N_PAD, 16), jnp.float32),
        ],
    )(src3_s, dst3_s, src3_t, dst3_t, n2d_s, n2d_t, z16)


def _agg_body(src3_s, dst3_s, src3_t, dst3_t, g_sl, g_sh, g_tl, g_th, z128,
              y_sl, y_sh, y_tl, y_th,
              isrc, idst, buf_a, buf_b, acc128, sem_a, sem_b):
    s = lax.axis_index("s")
    c = lax.axis_index("c")
    rows = pl.ds(s * ROWS_PER_TILE, ROWS_PER_TILE)

    def agg_phase(srcslab, dstslab, gref, yout):
        pltpu.sync_copy(z128.at[rows], acc128.at[rows])
        plsc.subcore_barrier()
        for ch in range(NCHUNK2):
            pltpu.sync_copy(srcslab.at[s, ch], isrc)
            pltpu.sync_copy(dstslab.at[s, ch], idst)
            pltpu.async_copy(gref.at[isrc.at[0]], buf_a, sem_a)

            def pair(i, carry):
                b0 = 2 * i
                pltpu.async_copy(gref.at[isrc.at[b0 + 1]], buf_b, sem_b)
                pltpu.make_async_copy(
                    gref.at[isrc.at[b0]], buf_a, sem_a).wait()
                pltpu.sync_copy(buf_a, acc128.at[idst.at[b0]], add=True)

                @pl.when(i < CB2 // 2 - 1)
                def _():
                    pltpu.async_copy(gref.at[isrc.at[b0 + 2]], buf_a, sem_a)

                pltpu.make_async_copy(
                    gref.at[isrc.at[b0 + 1]], buf_b, sem_b).wait()
                pltpu.sync_copy(buf_b, acc128.at[idst.at[b0 + 1]], add=True)
                return carry

            lax.fori_loop(0, CB2 // 2, pair, 0)
        plsc.subcore_barrier()
        pltpu.sync_copy(acc128.at[rows], yout.at[rows])
        plsc.subcore_barrier()

    @pl.when(c == 0)
    def _():
        agg_phase(src3_s, dst3_s, g_sl, y_sl)
        agg_phase(src3_t, dst3_t, g_tl, y_tl)

    @pl.when(c == 1)
    def _():
        agg_phase(src3_s, dst3_s, g_sh, y_sh)
        agg_phase(src3_t, dst3_t, g_th, y_th)


def _run_agg(src3_s, dst3_s, src3_t, dst3_t, g_sl, g_sh, g_tl, g_th, z128):
    y_sds = jax.ShapeDtypeStruct((N_PAD, D_HALF), jnp.float32)
    return pl.kernel(
        _agg_body,
        out_type=[y_sds, y_sds, y_sds, y_sds],
        mesh=plsc.VectorSubcoreMesh(core_axis_name="c", subcore_axis_name="s"),
        compiler_params=pltpu.CompilerParams(needs_layout_passes=False, use_tc_tiling_on_sc=False),
        scratch_types=[
            pltpu.VMEM((CB2, BATCH), jnp.int32),
            pltpu.VMEM((CB2, BATCH), jnp.int32),
            pltpu.VMEM((BATCH, D_HALF), jnp.float32),
            pltpu.VMEM((BATCH, D_HALF), jnp.float32),
            pltpu.VMEM_SHARED((N_PAD, D_HALF), jnp.float32),
            pltpu.SemaphoreType.DMA,
            pltpu.SemaphoreType.DMA,
        ],
    )(src3_s, dst3_s, src3_t, dst3_t, g_sl, g_sh, g_tl, g_th, z128)


def _mm_body(ylo_ref, yhi_ref, x_ref, n_ref, ss_ref, w1lo_ref, w1hi_ref,
             b1_ref, u_ref):
    i = pl.program_id(0)
    norm = n_ref[...][:, 0:1]
    ssum = ss_ref[...][:, 0:1]
    nsq = norm * norm
    x = x_ref[...]
    mlo = norm * ylo_ref[...] + nsq * x[:, :D_HALF]
    mhi = norm * yhi_ref[...] + nsq * x[:, D_HALF:]
    h1 = jnp.dot(mlo, w1lo_ref[...], preferred_element_type=jnp.float32,
                 precision=lax.Precision.HIGHEST)
    h1 = h1 + jnp.dot(mhi, w1hi_ref[...], preferred_element_type=jnp.float32,
                      precision=lax.Precision.HIGHEST)
    h1 = h1 + b1_ref[...]
    r = jnp.maximum(h1, 0.0)
    rowid = i * R_BLK + lax.broadcasted_iota(jnp.int32, (R_BLK, 1), 0)
    coef = jnp.where(rowid < N_NODES, norm * ssum + nsq, 0.0)
    part = jnp.sum(r * coef, axis=0, keepdims=True)

    @pl.when(i == 0)
    def _():
        u_ref[...] = part

    @pl.when(i != 0)
    def _():
        u_ref[...] = u_ref[...] + part


def _run_mm(ylo, yhi, xpad, norm16, ss16, w1lo, w1hi, b1r):
    return pl.pallas_call(
        _mm_body,
        grid=(NB,),
        in_specs=[
            pl.BlockSpec((R_BLK, D_HALF), lambda i: (i, 0)),
            pl.BlockSpec((R_BLK, D_HALF), lambda i: (i, 0)),
            pl.BlockSpec((R_BLK, D_IN), lambda i: (i, 0)),
            pl.BlockSpec((R_BLK, 16), lambda i: (i, 0)),
            pl.BlockSpec((R_BLK, 16), lambda i: (i, 0)),
            pl.BlockSpec((D_HALF, D_H), lambda i: (0, 0)),
            pl.BlockSpec((D_HALF, D_H), lambda i: (0, 0)),
            pl.BlockSpec((1, D_H), lambda i: (0, 0)),
        ],
        out_specs=pl.BlockSpec((1, D_H), lambda i: (0, 0)),
        out_shape=jax.ShapeDtypeStruct((1, D_H), jnp.float32),
    )(ylo, yhi, xpad, norm16, ss16, w1lo, w1hi, b1r)


def _head_body(u_ref, w2_ref, b2_ref, wq_ref, wk_ref, wv_ref, wo_ref,
               fc1w_ref, fc1b_ref, fc2w_ref, fc2b_ref, hm_ref, hmt_ref,
               out_ref):
    dot = functools.partial(jnp.dot, preferred_element_type=jnp.float32,
                            precision=lax.Precision.HIGHEST)
    z = dot(u_ref[...] * (1.0 / N_NODES), w2_ref[...]) + b2_ref[...]
    A = dot(z, wq_ref[...])
    B = dot(z, wk_ref[...])
    C = dot(z, wv_ref[...])
    hm = hm_ref[...]
    hmt = hmt_ref[...]
    sc = 1.0 / np.sqrt(HEAD_DIM).astype(np.float32)
    a0, a1 = A[0:1], A[1:2]
    b0, b1 = B[0:1], B[1:2]
    c0, c1 = C[0:1], C[1:2]
    l00 = dot(a0 * b0, hm) * sc
    l01 = dot(a0 * b1, hm) * sc
    l10 = dot(a1 * b0, hm) * sc
    l11 = dot(a1 * b1, hm) * sc
    p00 = 1.0 / (1.0 + jnp.exp(l01 - l00))
    p10 = 1.0 / (1.0 + jnp.exp(l11 - l10))
    o0 = dot(p00, hmt) * c0 + dot(1.0 - p00, hmt) * c1
    o1 = dot(p10, hmt) * c0 + dot(1.0 - p10, hmt) * c1
    o = jnp.concatenate([o0, o1], axis=0)
    e = dot(o, wo_ref[...])
    h = dot(e, fc1w_ref[...]) + fc1b_ref[...]
    h = jnp.where(h > 0, h, 0.01 * h)
    out_ref[...] = dot(h, fc2w_ref[...]) + fc2b_ref[...]


def _run_head(U, W2, b2r, Wq, Wk, Wv, Wo, fc1_W, fc1br, fc2_Wp, fc2_bp,
              Hmat, HmatT):
    return pl.pallas_call(
        _head_body,
        out_shape=jax.ShapeDtypeStruct((2, 128), jnp.float32),
    )(U, W2, b2r, Wq, Wk, Wv, Wo, fc1_W, fc1br, fc2_Wp, fc2_bp, Hmat, HmatT)


def kernel(x_s, edge_index_s, x_t, edge_index_t, W1, b1, W2, b2,
           Wq, Wk, Wv, Wo, fc1_W, fc1_b, fc2_W, fc2_b):
    f32 = jnp.float32
    ei_s = edge_index_s.astype(jnp.int32)
    ei_t = edge_index_t.astype(jnp.int32)

    def slabs(v):
        v = jnp.concatenate(
            [v, jnp.full((E_PAD - E_EDGES,), N_NODES, jnp.int32)])
        return v.reshape(NS, NCHUNK, CB, BATCH)

    src3_s, dst3_s = slabs(ei_s[0]), slabs(ei_s[1])
    src3_t, dst3_t = slabs(ei_t[0]), slabs(ei_t[1])

    xs_pad = jnp.pad(x_s, ((0, N_PAD - N_NODES), (0, 0)))
    xt_pad = jnp.pad(x_t, ((0, N_PAD - N_NODES), (0, 0)))

    ones128 = jnp.ones((BATCH, 16), f32)
    zeros16 = jnp.zeros((N_PAD, 16), f32)
    zeros128 = jnp.zeros((N_PAD, D_HALF), f32)

    deg16 = _run_deg(dst3_s, dst3_t, ones128, zeros16)
    norm16_s, g_s, norm2d_s = _run_norm_g(deg16[0], xs_pad)
    norm16_t, g_t, norm2d_t = _run_norm_g(deg16[1], xt_pad)

    ss_s, ss_t = _run_ssum(src3_s, dst3_s, src3_t, dst3_t,
                           norm2d_s, norm2d_t, zeros16)

    def wide(v):
        return v.reshape(NS, NCHUNK2, CB2, BATCH)

    y_sl, y_sh, y_tl, y_th = _run_agg(
        wide(src3_s), wide(dst3_s), wide(src3_t), wide(dst3_t),
        g_s[:, :D_HALF], g_s[:, D_HALF:],
        g_t[:, :D_HALF], g_t[:, D_HALF:], zeros128)

    w1lo, w1hi, b1r = W1[:D_HALF], W1[D_HALF:], b1.reshape(1, D_H)
    u_s = _run_mm(y_sl, y_sh, xs_pad, norm16_s, ss_s, w1lo, w1hi, b1r)
    u_t = _run_mm(y_tl, y_th, xt_pad, norm16_t, ss_t, w1lo, w1hi, b1r)
    U = jnp.concatenate([u_s, u_t], axis=0)

    Hmat = (lax.broadcasted_iota(jnp.int32, (D_H, N_HEADS), 0) // HEAD_DIM
            == lax.broadcasted_iota(jnp.int32, (D_H, N_HEADS), 1)).astype(f32)
    out = _run_head(U, W2, b2.reshape(1, D_H), Wq, Wk, Wv, Wo,
                    fc1_W, fc1_b.reshape(1, FC_DIM),
                    jnp.pad(fc2_W, ((0, 0), (0, 127))),
                    jnp.pad(fc2_b, (0, 127)).reshape(1, 128),
                    Hmat, Hmat.T)
    return out[:, :1]


# speedup vs baseline: 1.0605x; 1.0605x over previous
import functools

import jax
import jax.numpy as jnp
import numpy as np
from jax import lax
from jax.experimental import pallas as pl
from jax.experimental.pallas import tpu as pltpu
from jax.experimental.pallas import tpu_sc as plsc

N_NODES = 10000
N_PAD = 10240
E_EDGES = 160000
E_PAD = 163840
D_IN = 256
D_HALF = 128
D_H = 512
FC_DIM = 1024
N_HEADS = 8
HEAD_DIM = 64

NC = 2
NS = 16
BATCH = 128
NBATCH = E_PAD // (NS * BATCH)
CB = 8
NCHUNK = NBATCH // CB
CB2 = 40
NCHUNK2 = NBATCH // CB2
ROWS_PER_TILE = N_PAD // NS

R_BLK = 1024
NB = N_PAD // R_BLK


def _deg_body(dst_s, dst_t, ones_hbm, zeros16, deg_out, idx_v, ones_v, acc):
    c = lax.axis_index("c")
    s = lax.axis_index("s")
    rows = pl.ds(s * ROWS_PER_TILE, ROWS_PER_TILE)
    pltpu.sync_copy(ones_hbm, ones_v)
    pltpu.sync_copy(zeros16.at[rows], acc.at[rows])
    plsc.subcore_barrier()

    def count(slab):
        def ch_body(ch, carry):
            pltpu.sync_copy(slab.at[s, ch], idx_v)
            for b in range(CB):
                pltpu.sync_copy(ones_v, acc.at[idx_v.at[b]], add=True)
            return carry

        lax.fori_loop(0, NCHUNK, ch_body, 0)

    @pl.when(c == 0)
    def _():
        count(dst_s)

    @pl.when(c == 1)
    def _():
        count(dst_t)

    plsc.subcore_barrier()

    @pl.when(c == 0)
    def _():
        pltpu.sync_copy(acc.at[rows], deg_out.at[0, rows])

    @pl.when(c == 1)
    def _():
        pltpu.sync_copy(acc.at[rows], deg_out.at[1, rows])


def _run_deg(dst3_s, dst3_t, ones128, zeros16):
    return pl.kernel(
        _deg_body,
        out_type=jax.ShapeDtypeStruct((2, N_PAD, 16), jnp.float32),
        mesh=plsc.VectorSubcoreMesh(core_axis_name="c", subcore_axis_name="s"),
        compiler_params=pltpu.CompilerParams(needs_layout_passes=False, use_tc_tiling_on_sc=False),
        scratch_types=[
            pltpu.VMEM((CB, BATCH), jnp.int32),
            pltpu.VMEM((BATCH, 16), jnp.float32),
            pltpu.VMEM_SHARED((N_PAD, 16), jnp.float32),
        ],
    )(dst3_s, dst3_t, ones128, zeros16)


def _norm_g_body(deg_ref, x_ref, norm_ref, g_ref):
    deg = deg_ref[0]
    norm = lax.rsqrt(deg + 1.0)
    norm_ref[0] = norm
    g_ref[0] = x_ref[0] * norm[:, 0:1]


def _norm1d_body(deg2d_ref, norm2d_ref):
    norm2d_ref[...] = lax.rsqrt(deg2d_ref[...] + 1.0)


def _run_norm1d(deg2d):
    return pl.pallas_call(
        _norm1d_body,
        out_shape=jax.ShapeDtypeStruct(deg2d.shape, jnp.float32),
    )(deg2d)


def _run_norm_g(deg16, x2):
    return pl.pallas_call(
        _norm_g_body,
        grid=(2, NB),
        in_specs=[
            pl.BlockSpec((1, R_BLK, 16), lambda g, i: (g, i, 0)),
            pl.BlockSpec((1, R_BLK, D_IN), lambda g, i: (g, i, 0)),
        ],
        out_specs=[
            pl.BlockSpec((1, R_BLK, 16), lambda g, i: (g, i, 0)),
            pl.BlockSpec((1, R_BLK, D_IN), lambda g, i: (g, i, 0)),
        ],
        out_shape=[
            jax.ShapeDtypeStruct((2, N_PAD, 16), jnp.float32),
            jax.ShapeDtypeStruct((2, N_PAD, D_IN), jnp.float32),
        ],
    )(deg16, x2)


def _ssum_body(src3_s, dst3_s, src3_t, dst3_t, n2d_s, n2d_t, z16,
               ss_s, ss_t, isrc, idst, nbuf, nvm, acc16):
    s = lax.axis_index("s")
    c = lax.axis_index("c")
    rows = pl.ds(s * ROWS_PER_TILE, ROWS_PER_TILE)

    def ssum_phase(srcslab, dstslab, n2d, out):
        pltpu.sync_copy(n2d, nvm)
        pltpu.sync_copy(z16.at[pl.ds(0, BATCH)], nbuf)
        pltpu.sync_copy(z16.at[rows], acc16.at[rows])
        plsc.subcore_barrier()
        lane = lax.iota(jnp.int32, 16)
        col0 = jnp.zeros((16,), jnp.int32)

        def ch_body(ch, carry):
            pltpu.sync_copy(srcslab.at[s, ch], isrc)
            pltpu.sync_copy(dstslab.at[s, ch], idst)
            for b in range(CB):
                for k in range(BATCH // 16):
                    dvec = idst[b, pl.ds(k * 16, 16)]
                    vals = plsc.load_gather(
                        nvm, [lax.shift_right_logical(dvec, 7),
                              lax.bitwise_and(dvec, 127)])
                    plsc.store_scatter(nbuf, [lane + (k * 16), col0], vals)
                pltpu.sync_copy(nbuf, acc16.at[isrc.at[b]], add=True)
            return carry

        lax.fori_loop(0, NCHUNK, ch_body, 0)
        plsc.subcore_barrier()
        pltpu.sync_copy(acc16.at[rows], out.at[rows])

    @pl.when(c == 0)
    def _():
        ssum_phase(src3_s, dst3_s, n2d_s, ss_s)

    @pl.when(c == 1)
    def _():
        ssum_phase(src3_t, dst3_t, n2d_t, ss_t)


def _run_ssum(src3_s, dst3_s, src3_t, dst3_t, n2d_s, n2d_t, z16):
    s_sds = jax.ShapeDtypeStruct((N_PAD, 16), jnp.float32)
    return pl.kernel(
        _ssum_body,
        out_type=[s_sds, s_sds],
        mesh=plsc.VectorSubcoreMesh(core_axis_name="c", subcore_axis_name="s"),
        compiler_params=pltpu.CompilerParams(needs_layout_passes=False, use_tc_tiling_on_sc=False),
        scratch_types=[
            pltpu.VMEM((CB, BATCH), jnp.int32),
            pltpu.VMEM((CB, BATCH), jnp.int32),
            pltpu.VMEM((BATCH, 16), jnp.float32),
            pltpu.VMEM((N_PAD // 128, 128), jnp.float32),
            pltpu.VMEM_SHARED((N_PAD, 16), jnp.float32),
        ],
    )(src3_s, dst3_s, src3_t, dst3_t, n2d_s, n2d_t, z16)


def _agg_body(src3_s, dst3_s, src3_t, dst3_t, g_sl, g_sh, g_tl, g_th, z128,
              y_sl, y_sh, y_tl, y_th,
              isrc, idst, buf_a, buf_b, acc128, sem_a, sem_b):
    s = lax.axis_index("s")
    c = lax.axis_index("c")
    rows = pl.ds(s * ROWS_PER_TILE, ROWS_PER_TILE)

    def agg_phase(srcslab, dstslab, gref, yout):
        pltpu.sync_copy(z128.at[rows], acc128.at[rows])
        plsc.subcore_barrier()
        for ch in range(NCHUNK2):
            pltpu.sync_copy(srcslab.at[s, ch], isrc)
            pltpu.sync_copy(dstslab.at[s, ch], idst)
            pltpu.async_copy(gref.at[isrc.at[0]], buf_a, sem_a)

            def pair(i, carry):
                b0 = 2 * i
                pltpu.async_copy(gref.at[isrc.at[b0 + 1]], buf_b, sem_b)
                pltpu.make_async_copy(
                    gref.at[isrc.at[b0]], buf_a, sem_a).wait()
                pltpu.sync_copy(buf_a, acc128.at[idst.at[b0]], add=True)

                @pl.when(i < CB2 // 2 - 1)
                def _():
                    pltpu.async_copy(gref.at[isrc.at[b0 + 2]], buf_a, sem_a)

                pltpu.make_async_copy(
                    gref.at[isrc.at[b0 + 1]], buf_b, sem_b).wait()
                pltpu.sync_copy(buf_b, acc128.at[idst.at[b0 + 1]], add=True)
                return carry

            lax.fori_loop(0, CB2 // 2, pair, 0)
        plsc.subcore_barrier()
        pltpu.sync_copy(acc128.at[rows], yout.at[rows])
        plsc.subcore_barrier()

    @pl.when(c == 0)
    def _():
        agg_phase(src3_s, dst3_s, g_sl, y_sl)
        agg_phase(src3_t, dst3_t, g_tl, y_tl)

    @pl.when(c == 1)
    def _():
        agg_phase(src3_s, dst3_s, g_sh, y_sh)
        agg_phase(src3_t, dst3_t, g_th, y_th)


def _run_agg(src3_s, dst3_s, src3_t, dst3_t, g_sl, g_sh, g_tl, g_th, z128):
    y_sds = jax.ShapeDtypeStruct((N_PAD, D_HALF), jnp.float32)
    return pl.kernel(
        _agg_body,
        out_type=[y_sds, y_sds, y_sds, y_sds],
        mesh=plsc.VectorSubcoreMesh(core_axis_name="c", subcore_axis_name="s"),
        compiler_params=pltpu.CompilerParams(needs_layout_passes=False, use_tc_tiling_on_sc=False),
        scratch_types=[
            pltpu.VMEM((CB2, BATCH), jnp.int32),
            pltpu.VMEM((CB2, BATCH), jnp.int32),
            pltpu.VMEM((BATCH, D_HALF), jnp.float32),
            pltpu.VMEM((BATCH, D_HALF), jnp.float32),
            pltpu.VMEM_SHARED((N_PAD, D_HALF), jnp.float32),
            pltpu.SemaphoreType.DMA,
            pltpu.SemaphoreType.DMA,
        ],
    )(src3_s, dst3_s, src3_t, dst3_t, g_sl, g_sh, g_tl, g_th, z128)


def _mm_body(ylo_ref, yhi_ref, x_ref, n_ref, ss_ref, w1lo_ref, w1hi_ref,
             b1_ref, u_ref):
    i = pl.program_id(1)
    norm = n_ref[0][:, 0:1]
    ssum = ss_ref[0][:, 0:1]
    nsq = norm * norm
    x = x_ref[0]
    mlo = norm * ylo_ref[0] + nsq * x[:, :D_HALF]
    mhi = norm * yhi_ref[0] + nsq * x[:, D_HALF:]
    h1 = jnp.dot(mlo, w1lo_ref[...], preferred_element_type=jnp.float32,
                 precision=lax.Precision.HIGHEST)
    h1 = h1 + jnp.dot(mhi, w1hi_ref[...], preferred_element_type=jnp.float32,
                      precision=lax.Precision.HIGHEST)
    h1 = h1 + b1_ref[...]
    r = jnp.maximum(h1, 0.0)
    rowid = i * R_BLK + lax.broadcasted_iota(jnp.int32, (R_BLK, 1), 0)
    coef = jnp.where(rowid < N_NODES, norm * ssum + nsq, 0.0)
    part = jnp.sum(r * coef, axis=0, keepdims=True)

    @pl.when(i == 0)
    def _():
        u_ref[0] = part

    @pl.when(i != 0)
    def _():
        u_ref[0] = u_ref[0] + part


def _run_mm(ylo, yhi, x2, norm16, ss16, w1lo, w1hi, b1r):
    return pl.pallas_call(
        _mm_body,
        grid=(2, NB),
        in_specs=[
            pl.BlockSpec((1, R_BLK, D_HALF), lambda g, i: (g, i, 0)),
            pl.BlockSpec((1, R_BLK, D_HALF), lambda g, i: (g, i, 0)),
            pl.BlockSpec((1, R_BLK, D_IN), lambda g, i: (g, i, 0)),
            pl.BlockSpec((1, R_BLK, 16), lambda g, i: (g, i, 0)),
            pl.BlockSpec((1, R_BLK, 16), lambda g, i: (g, i, 0)),
            pl.BlockSpec((D_HALF, D_H), lambda g, i: (0, 0)),
            pl.BlockSpec((D_HALF, D_H), lambda g, i: (0, 0)),
            pl.BlockSpec((1, D_H), lambda g, i: (0, 0)),
        ],
        out_specs=pl.BlockSpec((1, 1, D_H), lambda g, i: (g, 0, 0)),
        out_shape=jax.ShapeDtypeStruct((2, 1, D_H), jnp.float32),
    )(ylo, yhi, x2, norm16, ss16, w1lo, w1hi, b1r)


def _head_body(u_ref, w2_ref, b2_ref, wq_ref, wk_ref, wv_ref, wo_ref,
               fc1w_ref, fc1b_ref, fc2w_ref, fc2b_ref, hm_ref, hmt_ref,
               out_ref):
    dot = functools.partial(jnp.dot, preferred_element_type=jnp.float32,
                            precision=lax.Precision.HIGHEST)
    z = dot(u_ref[...] * (1.0 / N_NODES), w2_ref[...]) + b2_ref[...]
    A = dot(z, wq_ref[...])
    B = dot(z, wk_ref[...])
    C = dot(z, wv_ref[...])
    hm = hm_ref[...]
    hmt = hmt_ref[...]
    sc = 1.0 / np.sqrt(HEAD_DIM).astype(np.float32)
    a0, a1 = A[0:1], A[1:2]
    b0, b1 = B[0:1], B[1:2]
    c0, c1 = C[0:1], C[1:2]
    l00 = dot(a0 * b0, hm) * sc
    l01 = dot(a0 * b1, hm) * sc
    l10 = dot(a1 * b0, hm) * sc
    l11 = dot(a1 * b1, hm) * sc
    p00 = 1.0 / (1.0 + jnp.exp(l01 - l00))
    p10 = 1.0 / (1.0 + jnp.exp(l11 - l10))
    o0 = dot(p00, hmt) * c0 + dot(1.0 - p00, hmt) * c1
    o1 = dot(p10, hmt) * c0 + dot(1.0 - p10, hmt) * c1
    o = jnp.concatenate([o0, o1], axis=0)
    e = dot(o, wo_ref[...])
    h = dot(e, fc1w_ref[...]) + fc1b_ref[...]
    h = jnp.where(h > 0, h, 0.01 * h)
    out_ref[...] = dot(h, fc2w_ref[...]) + fc2b_ref[...]


def _run_head(U, W2, b2r, Wq, Wk, Wv, Wo, fc1_W, fc1br, fc2_Wp, fc2_bp,
              Hmat, HmatT):
    return pl.pallas_call(
        _head_body,
        out_shape=jax.ShapeDtypeStruct((2, 128), jnp.float32),
    )(U, W2, b2r, Wq, Wk, Wv, Wo, fc1_W, fc1br, fc2_Wp, fc2_bp, Hmat, HmatT)


def kernel(x_s, edge_index_s, x_t, edge_index_t, W1, b1, W2, b2,
           Wq, Wk, Wv, Wo, fc1_W, fc1_b, fc2_W, fc2_b):
    f32 = jnp.float32
    ei_s = edge_index_s.astype(jnp.int32)
    ei_t = edge_index_t.astype(jnp.int32)

    def slabs(v):
        v = jnp.concatenate(
            [v, jnp.full((E_PAD - E_EDGES,), N_NODES, jnp.int32)])
        return v.reshape(NS, NCHUNK, CB, BATCH)

    src3_s, dst3_s = slabs(ei_s[0]), slabs(ei_s[1])
    src3_t, dst3_t = slabs(ei_t[0]), slabs(ei_t[1])

    xs_pad = jnp.pad(x_s, ((0, N_PAD - N_NODES), (0, 0)))
    xt_pad = jnp.pad(x_t, ((0, N_PAD - N_NODES), (0, 0)))
    x2 = jnp.stack([xs_pad, xt_pad])

    ones128 = jnp.ones((BATCH, 16), f32)
    zeros16 = jnp.zeros((N_PAD, 16), f32)
    zeros128 = jnp.zeros((N_PAD, D_HALF), f32)

    deg16 = _run_deg(dst3_s, dst3_t, ones128, zeros16)
    norm16, g2 = _run_norm_g(deg16, x2)
    norm2d = _run_norm1d(deg16[:, :, 0].reshape(2, N_PAD // 128, 128))

    ss_s, ss_t = _run_ssum(src3_s, dst3_s, src3_t, dst3_t,
                           norm2d[0], norm2d[1], zeros16)

    def wide(v):
        return v.reshape(NS, NCHUNK2, CB2, BATCH)

    y_sl, y_sh, y_tl, y_th = _run_agg(
        wide(src3_s), wide(dst3_s), wide(src3_t), wide(dst3_t),
        g2[0, :, :D_HALF], g2[0, :, D_HALF:],
        g2[1, :, :D_HALF], g2[1, :, D_HALF:], zeros128)

    ylo = jnp.stack([y_sl, y_tl])
    yhi = jnp.stack([y_sh, y_th])
    ss16 = jnp.stack([ss_s, ss_t])

    U = _run_mm(ylo, yhi, x2, norm16, ss16,
                W1[:D_HALF], W1[D_HALF:], b1.reshape(1, D_H)).reshape(2, D_H)

    Hmat = (lax.broadcasted_iota(jnp.int32, (D_H, N_HEADS), 0) // HEAD_DIM
            == lax.broadcasted_iota(jnp.int32, (D_H, N_HEADS), 1)).astype(f32)
    out = _run_head(U, W2, b2.reshape(1, D_H), Wq, Wk, Wv, Wo,
                    fc1_W, fc1_b.reshape(1, FC_DIM),
                    jnp.pad(fc2_W, ((0, 0), (0, 127))),
                    jnp.pad(fc2_b, (0, 127)).reshape(1, 128),
                    Hmat, Hmat.T)
    return out[:, :1]


# speedup vs baseline: 1.1261x; 1.0618x over previous
import functools

import jax
import jax.numpy as jnp
import numpy as np
from jax import lax
from jax.experimental import pallas as pl
from jax.experimental.pallas import tpu as pltpu
from jax.experimental.pallas import tpu_sc as plsc

N_NODES = 10000
N_PAD = 10240
E_EDGES = 160000
E_PAD = 163840
D_IN = 256
D_HALF = 128
D_H = 512
FC_DIM = 1024
N_HEADS = 8
HEAD_DIM = 64

NC = 2
NS = 16
BATCH = 128
NBATCH = E_PAD // (NS * BATCH)
CB = 8
NCHUNK = NBATCH // CB
CB2 = 40
NCHUNK2 = NBATCH // CB2
ROWS_PER_TILE = N_PAD // NS

R_BLK = 1024
NB = N_PAD // R_BLK


def _deg_body(dst_s, dst_t, ones_hbm, zeros16, deg_out, idx_v, ones_v, acc):
    c = lax.axis_index("c")
    s = lax.axis_index("s")
    rows = pl.ds(s * ROWS_PER_TILE, ROWS_PER_TILE)
    pltpu.sync_copy(ones_hbm, ones_v)
    pltpu.sync_copy(zeros16.at[rows], acc.at[rows])
    plsc.subcore_barrier()

    def count(slab):
        def ch_body(ch, carry):
            pltpu.sync_copy(slab.at[s, ch], idx_v)
            for b in range(CB):
                pltpu.sync_copy(ones_v, acc.at[idx_v.at[b]], add=True)
            return carry

        lax.fori_loop(0, NCHUNK, ch_body, 0)

    @pl.when(c == 0)
    def _():
        count(dst_s)

    @pl.when(c == 1)
    def _():
        count(dst_t)

    plsc.subcore_barrier()

    @pl.when(c == 0)
    def _():
        pltpu.sync_copy(acc.at[rows], deg_out.at[0, rows])

    @pl.when(c == 1)
    def _():
        pltpu.sync_copy(acc.at[rows], deg_out.at[1, rows])


def _run_deg(dst3_s, dst3_t, ones128, zeros16):
    return pl.kernel(
        _deg_body,
        out_type=jax.ShapeDtypeStruct((2, N_PAD, 16), jnp.float32),
        mesh=plsc.VectorSubcoreMesh(core_axis_name="c", subcore_axis_name="s"),
        compiler_params=pltpu.CompilerParams(needs_layout_passes=False, use_tc_tiling_on_sc=False),
        scratch_types=[
            pltpu.VMEM((CB, BATCH), jnp.int32),
            pltpu.VMEM((BATCH, 16), jnp.float32),
            pltpu.VMEM_SHARED((N_PAD, 16), jnp.float32),
        ],
    )(dst3_s, dst3_t, ones128, zeros16)


def _norm_g_body(deg_ref, x_ref, norm_ref, g_ref):
    deg = deg_ref[0]
    norm = lax.rsqrt(deg + 1.0)
    norm_ref[0] = norm
    g_ref[0] = x_ref[0] * norm[:, 0:1]


def _norm1d_body(deg2d_ref, norm2d_ref):
    norm2d_ref[...] = lax.rsqrt(deg2d_ref[...] + 1.0)


def _run_norm1d(deg2d):
    return pl.pallas_call(
        _norm1d_body,
        out_shape=jax.ShapeDtypeStruct(deg2d.shape, jnp.float32),
    )(deg2d)


def _run_norm_g(deg16, x2):
    return pl.pallas_call(
        _norm_g_body,
        grid=(2, NB),
        in_specs=[
            pl.BlockSpec((1, R_BLK, 16), lambda g, i: (g, i, 0)),
            pl.BlockSpec((1, R_BLK, D_IN), lambda g, i: (g, i, 0)),
        ],
        out_specs=[
            pl.BlockSpec((1, R_BLK, 16), lambda g, i: (g, i, 0)),
            pl.BlockSpec((1, R_BLK, D_IN), lambda g, i: (g, i, 0)),
        ],
        out_shape=[
            jax.ShapeDtypeStruct((2, N_PAD, 16), jnp.float32),
            jax.ShapeDtypeStruct((2, N_PAD, D_IN), jnp.float32),
        ],
    )(deg16, x2)


def _ssum_body(src3_s, dst3_s, src3_t, dst3_t, n2d_s, n2d_t, z16,
               ss_s, ss_t, isrc, idst, nbuf, nvm, acc16):
    s = lax.axis_index("s")
    c = lax.axis_index("c")
    rows = pl.ds(s * ROWS_PER_TILE, ROWS_PER_TILE)

    def ssum_phase(srcslab, dstslab, n2d, out):
        pltpu.sync_copy(n2d, nvm)
        pltpu.sync_copy(z16.at[pl.ds(0, BATCH)], nbuf)
        pltpu.sync_copy(z16.at[rows], acc16.at[rows])
        plsc.subcore_barrier()
        lane = lax.iota(jnp.int32, 16)
        col0 = jnp.zeros((16,), jnp.int32)

        def ch_body(ch, carry):
            pltpu.sync_copy(srcslab.at[s, ch], isrc)
            pltpu.sync_copy(dstslab.at[s, ch], idst)
            for b in range(CB):
                for k in range(BATCH // 16):
                    dvec = idst[b, pl.ds(k * 16, 16)]
                    vals = plsc.load_gather(
                        nvm, [lax.shift_right_logical(dvec, 7),
                              lax.bitwise_and(dvec, 127)])
                    plsc.store_scatter(nbuf, [lane + (k * 16), col0], vals)
                pltpu.sync_copy(nbuf, acc16.at[isrc.at[b]], add=True)
            return carry

        lax.fori_loop(0, NCHUNK, ch_body, 0)
        plsc.subcore_barrier()
        pltpu.sync_copy(acc16.at[rows], out.at[rows])

    @pl.when(c == 0)
    def _():
        ssum_phase(src3_s, dst3_s, n2d_s, ss_s)

    @pl.when(c == 1)
    def _():
        ssum_phase(src3_t, dst3_t, n2d_t, ss_t)


def _run_ssum(src3_s, dst3_s, src3_t, dst3_t, n2d_s, n2d_t, z16):
    s_sds = jax.ShapeDtypeStruct((N_PAD, 16), jnp.float32)
    return pl.kernel(
        _ssum_body,
        out_type=[s_sds, s_sds],
        mesh=plsc.VectorSubcoreMesh(core_axis_name="c", subcore_axis_name="s"),
        compiler_params=pltpu.CompilerParams(needs_layout_passes=False, use_tc_tiling_on_sc=False),
        scratch_types=[
            pltpu.VMEM((CB, BATCH), jnp.int32),
            pltpu.VMEM((CB, BATCH), jnp.int32),
            pltpu.VMEM((BATCH, 16), jnp.float32),
            pltpu.VMEM((N_PAD // 128, 128), jnp.float32),
            pltpu.VMEM_SHARED((N_PAD, 16), jnp.float32),
        ],
    )(src3_s, dst3_s, src3_t, dst3_t, n2d_s, n2d_t, z16)


def _agg_body(src3_s, dst3_s, src3_t, dst3_t, g_sl, g_sh, g_tl, g_th, z128,
              y_sl, y_sh, y_tl, y_th,
              isrc, idst, buf_a, buf_b, acc128, sem_a, sem_b):
    s = lax.axis_index("s")
    c = lax.axis_index("c")
    rows = pl.ds(s * ROWS_PER_TILE, ROWS_PER_TILE)

    def agg_phase(srcslab, dstslab, gref, yout):
        pltpu.sync_copy(z128.at[rows], acc128.at[rows])
        plsc.subcore_barrier()
        for ch in range(NCHUNK2):
            pltpu.sync_copy(srcslab.at[s, ch], isrc)
            pltpu.sync_copy(dstslab.at[s, ch], idst)
            pltpu.async_copy(gref.at[isrc.at[0]], buf_a, sem_a)

            def pair(i, carry):
                b0 = 2 * i
                pltpu.async_copy(gref.at[isrc.at[b0 + 1]], buf_b, sem_b)
                pltpu.make_async_copy(
                    gref.at[isrc.at[b0]], buf_a, sem_a).wait()
                pltpu.sync_copy(buf_a, acc128.at[idst.at[b0]], add=True)

                @pl.when(i < CB2 // 2 - 1)
                def _():
                    pltpu.async_copy(gref.at[isrc.at[b0 + 2]], buf_a, sem_a)

                pltpu.make_async_copy(
                    gref.at[isrc.at[b0 + 1]], buf_b, sem_b).wait()
                pltpu.sync_copy(buf_b, acc128.at[idst.at[b0 + 1]], add=True)
                return carry

            lax.fori_loop(0, CB2 // 2, pair, 0)
        plsc.subcore_barrier()
        pltpu.sync_copy(acc128.at[rows], yout.at[rows])
        plsc.subcore_barrier()

    @pl.when(c == 0)
    def _():
        agg_phase(src3_s, dst3_s, g_sl, y_sl)
        agg_phase(src3_t, dst3_t, g_tl, y_tl)

    @pl.when(c == 1)
    def _():
        agg_phase(src3_s, dst3_s, g_sh, y_sh)
        agg_phase(src3_t, dst3_t, g_th, y_th)


def _run_agg(src3_s, dst3_s, src3_t, dst3_t, g_sl, g_sh, g_tl, g_th, z128):
    y_sds = jax.ShapeDtypeStruct((N_PAD, D_HALF), jnp.float32)
    return pl.kernel(
        _agg_body,
        out_type=[y_sds, y_sds, y_sds, y_sds],
        mesh=plsc.VectorSubcoreMesh(core_axis_name="c", subcore_axis_name="s"),
        compiler_params=pltpu.CompilerParams(needs_layout_passes=False, use_tc_tiling_on_sc=False),
        scratch_types=[
            pltpu.VMEM((CB2, BATCH), jnp.int32),
            pltpu.VMEM((CB2, BATCH), jnp.int32),
            pltpu.VMEM((BATCH, D_HALF), jnp.float32),
            pltpu.VMEM((BATCH, D_HALF), jnp.float32),
            pltpu.VMEM_SHARED((N_PAD, D_HALF), jnp.float32),
            pltpu.SemaphoreType.DMA,
            pltpu.SemaphoreType.DMA,
        ],
    )(src3_s, dst3_s, src3_t, dst3_t, g_sl, g_sh, g_tl, g_th, z128)


def _mm_body(ylo_ref, yhi_ref, x_ref, n_ref, ss_ref, w1lo_ref, w1hi_ref,
             b1_ref, u_ref):
    i = pl.program_id(1)
    norm = n_ref[0][:, 0:1]
    ssum = ss_ref[0][:, 0:1]
    nsq = norm * norm
    x = x_ref[0]
    mlo = norm * ylo_ref[0] + nsq * x[:, :D_HALF]
    mhi = norm * yhi_ref[0] + nsq * x[:, D_HALF:]
    h1 = jnp.dot(mlo, w1lo_ref[...], preferred_element_type=jnp.float32)
    h1 = h1 + jnp.dot(mhi, w1hi_ref[...], preferred_element_type=jnp.float32)
    h1 = h1 + b1_ref[...]
    r = jnp.maximum(h1, 0.0)
    rowid = i * R_BLK + lax.broadcasted_iota(jnp.int32, (R_BLK, 1), 0)
    coef = jnp.where(rowid < N_NODES, norm * ssum + nsq, 0.0)
    part = jnp.sum(r * coef, axis=0, keepdims=True)

    @pl.when(i == 0)
    def _():
        u_ref[0] = part

    @pl.when(i != 0)
    def _():
        u_ref[0] = u_ref[0] + part


def _run_mm(ylo, yhi, x2, norm16, ss16, w1lo, w1hi, b1r):
    return pl.pallas_call(
        _mm_body,
        grid=(2, NB),
        in_specs=[
            pl.BlockSpec((1, R_BLK, D_HALF), lambda g, i: (g, i, 0)),
            pl.BlockSpec((1, R_BLK, D_HALF), lambda g, i: (g, i, 0)),
            pl.BlockSpec((1, R_BLK, D_IN), lambda g, i: (g, i, 0)),
            pl.BlockSpec((1, R_BLK, 16), lambda g, i: (g, i, 0)),
            pl.BlockSpec((1, R_BLK, 16), lambda g, i: (g, i, 0)),
            pl.BlockSpec((D_HALF, D_H), lambda g, i: (0, 0)),
            pl.BlockSpec((D_HALF, D_H), lambda g, i: (0, 0)),
            pl.BlockSpec((1, D_H), lambda g, i: (0, 0)),
        ],
        out_specs=pl.BlockSpec((1, 1, D_H), lambda g, i: (g, 0, 0)),
        out_shape=jax.ShapeDtypeStruct((2, 1, D_H), jnp.float32),
    )(ylo, yhi, x2, norm16, ss16, w1lo, w1hi, b1r)


def _head_body(u_ref, w2_ref, b2_ref, wq_ref, wk_ref, wv_ref, wo_ref,
               fc1w_ref, fc1b_ref, fc2w_ref, fc2b_ref, hm_ref, hmt_ref,
               out_ref):
    dot = functools.partial(jnp.dot, preferred_element_type=jnp.float32)
    z = dot(u_ref[...] * (1.0 / N_NODES), w2_ref[...]) + b2_ref[...]
    A = dot(z, wq_ref[...])
    B = dot(z, wk_ref[...])
    C = dot(z, wv_ref[...])
    hm = hm_ref[...]
    hmt = hmt_ref[...]
    sc = 1.0 / np.sqrt(HEAD_DIM).astype(np.float32)
    a0, a1 = A[0:1], A[1:2]
    b0, b1 = B[0:1], B[1:2]
    c0, c1 = C[0:1], C[1:2]
    l00 = dot(a0 * b0, hm) * sc
    l01 = dot(a0 * b1, hm) * sc
    l10 = dot(a1 * b0, hm) * sc
    l11 = dot(a1 * b1, hm) * sc
    p00 = 1.0 / (1.0 + jnp.exp(l01 - l00))
    p10 = 1.0 / (1.0 + jnp.exp(l11 - l10))
    o0 = dot(p00, hmt) * c0 + dot(1.0 - p00, hmt) * c1
    o1 = dot(p10, hmt) * c0 + dot(1.0 - p10, hmt) * c1
    o = jnp.concatenate([o0, o1], axis=0)
    e = dot(o, wo_ref[...])
    h = dot(e, fc1w_ref[...]) + fc1b_ref[...]
    h = jnp.where(h > 0, h, 0.01 * h)
    out_ref[...] = dot(h, fc2w_ref[...]) + fc2b_ref[...]


def _run_head(U, W2, b2r, Wq, Wk, Wv, Wo, fc1_W, fc1br, fc2_Wp, fc2_bp,
              Hmat, HmatT):
    return pl.pallas_call(
        _head_body,
        out_shape=jax.ShapeDtypeStruct((2, 128), jnp.float32),
    )(U, W2, b2r, Wq, Wk, Wv, Wo, fc1_W, fc1br, fc2_Wp, fc2_bp, Hmat, HmatT)


def kernel(x_s, edge_index_s, x_t, edge_index_t, W1, b1, W2, b2,
           Wq, Wk, Wv, Wo, fc1_W, fc1_b, fc2_W, fc2_b):
    f32 = jnp.float32
    ei_s = edge_index_s.astype(jnp.int32)
    ei_t = edge_index_t.astype(jnp.int32)

    def slabs(v):
        v = jnp.concatenate(
            [v, jnp.full((E_PAD - E_EDGES,), N_NODES, jnp.int32)])
        return v.reshape(NS, NCHUNK, CB, BATCH)

    src3_s, dst3_s = slabs(ei_s[0]), slabs(ei_s[1])
    src3_t, dst3_t = slabs(ei_t[0]), slabs(ei_t[1])

    xs_pad = jnp.pad(x_s, ((0, N_PAD - N_NODES), (0, 0)))
    xt_pad = jnp.pad(x_t, ((0, N_PAD - N_NODES), (0, 0)))
    x2 = jnp.stack([xs_pad, xt_pad])

    ones128 = jnp.ones((BATCH, 16), f32)
    zeros16 = jnp.zeros((N_PAD, 16), f32)
    zeros128 = jnp.zeros((N_PAD, D_HALF), f32)

    deg16 = _run_deg(dst3_s, dst3_t, ones128, zeros16)
    norm16, g2 = _run_norm_g(deg16, x2)
    norm2d = _run_norm1d(deg16[:, :, 0].reshape(2, N_PAD // 128, 128))

    ss_s, ss_t = _run_ssum(src3_s, dst3_s, src3_t, dst3_t,
                           norm2d[0], norm2d[1], zeros16)

    def wide(v):
        return v.reshape(NS, NCHUNK2, CB2, BATCH)

    y_sl, y_sh, y_tl, y_th = _run_agg(
        wide(src3_s), wide(dst3_s), wide(src3_t), wide(dst3_t),
        g2[0, :, :D_HALF], g2[0, :, D_HALF:],
        g2[1, :, :D_HALF], g2[1, :, D_HALF:], zeros128)

    ylo = jnp.stack([y_sl, y_tl])
    yhi = jnp.stack([y_sh, y_th])
    ss16 = jnp.stack([ss_s, ss_t])

    U = _run_mm(ylo, yhi, x2, norm16, ss16,
                W1[:D_HALF], W1[D_HALF:], b1.reshape(1, D_H)).reshape(2, D_H)

    Hmat = (lax.broadcasted_iota(jnp.int32, (D_H, N_HEADS), 0) // HEAD_DIM
            == lax.broadcasted_iota(jnp.int32, (D_H, N_HEADS), 1)).astype(f32)
    out = _run_head(U, W2, b2.reshape(1, D_H), Wq, Wk, Wv, Wo,
                    fc1_W, fc1_b.reshape(1, FC_DIM),
                    jnp.pad(fc2_W, ((0, 0), (0, 127))),
                    jnp.pad(fc2_b, (0, 127)).reshape(1, 128),
                    Hmat, Hmat.T)
    return out[:, :1]
